# Initial kernel scaffold; baseline (speedup 1.0000x reference)
import jax
import jax.numpy as jnp
from jax.experimental import pallas as pl


def kernel(logits, gumbel_noise, rewards):
    raise NotImplementedError("write your pallas kernel here")


# speedup vs baseline: 1.9242x; 1.9242x over previous
import functools

import jax
import jax.numpy as jnp
from jax.experimental import pallas as pl
from jax.experimental.pallas import tpu as pltpu

BATCH_ = 32
VOCAB_ = 1_000_000
VBLK = 16_384
GRID = -(-VOCAB_ // VBLK)

_NEG_INF = float("-inf")


def _body(logits_ref, gumbel_ref, rewards_ref,
          loss_ref, actions_ref,
          m_ref, acc_ref, gm_ref, gi_ref, gl_ref):
    j = pl.program_id(0)

    @pl.when(j == 0)
    def _init():
        m_ref[...] = jnp.full((BATCH_, 1), _NEG_INF, jnp.float32)
        acc_ref[...] = jnp.zeros((BATCH_, 1), jnp.float32)
        gm_ref[...] = jnp.full((BATCH_, 1), _NEG_INF, jnp.float32)
        gi_ref[...] = jnp.zeros((BATCH_, 1), jnp.int32)
        gl_ref[...] = jnp.zeros((BATCH_, 1), jnp.float32)

    iota = jax.lax.broadcasted_iota(jnp.int32, (BATCH_, VBLK), 1)
    valid = (j * VBLK + iota) < VOCAB_
    x = jnp.where(valid, logits_ref[...], _NEG_INF)
    u = jnp.where(valid, gumbel_ref[...], 0.5)

    bm = jnp.max(x, axis=1, keepdims=True)
    m_old = m_ref[...]
    m_new = jnp.maximum(m_old, bm)
    acc_ref[...] = (acc_ref[...] * jnp.exp(m_old - m_new)
                    + jnp.sum(jnp.exp(x - m_new), axis=1, keepdims=True))
    m_ref[...] = m_new

    s = x - jnp.log(-jnp.log(u))
    lm = jnp.max(s, axis=1, keepdims=True)
    big = jnp.int32(2**31 - 1)
    li = jnp.min(jnp.where(s == lm, iota, big), axis=1, keepdims=True)
    lx = jnp.sum(jnp.where(iota == li, x, 0.0), axis=1, keepdims=True)

    better = lm > gm_ref[...]
    gi_ref[...] = jnp.where(better, j * VBLK + li, gi_ref[...])
    gl_ref[...] = jnp.where(better, lx, gl_ref[...])
    gm_ref[...] = jnp.maximum(gm_ref[...], lm)

    @pl.when(j == GRID - 1)
    def _fini():
        lse = m_ref[...] + jnp.log(acc_ref[...])
        log_p = gl_ref[...] - lse
        r = rewards_ref[...]
        loss_ref[...] = jnp.sum(-log_p * r, keepdims=True).reshape(1, 1) / BATCH_
        actions_ref[...] = gi_ref[...]


@jax.jit
def kernel(logits, gumbel_noise, rewards):
    rewards2 = rewards.reshape(BATCH_, 1)
    loss, actions = pl.pallas_call(
        _body,
        grid=(GRID,),
        in_specs=[
            pl.BlockSpec((BATCH_, VBLK), lambda j: (0, j)),
            pl.BlockSpec((BATCH_, VBLK), lambda j: (0, j)),
            pl.BlockSpec((BATCH_, 1), lambda j: (0, 0)),
        ],
        out_specs=[
            pl.BlockSpec((1, 1), lambda j: (0, 0)),
            pl.BlockSpec((BATCH_, 1), lambda j: (0, 0)),
        ],
        out_shape=[
            jax.ShapeDtypeStruct((1, 1), jnp.float32),
            jax.ShapeDtypeStruct((BATCH_, 1), jnp.int32),
        ],
        scratch_shapes=[
            pltpu.VMEM((BATCH_, 1), jnp.float32),
            pltpu.VMEM((BATCH_, 1), jnp.float32),
            pltpu.VMEM((BATCH_, 1), jnp.float32),
            pltpu.VMEM((BATCH_, 1), jnp.int32),
            pltpu.VMEM((BATCH_, 1), jnp.float32),
        ],
    )(logits, gumbel_noise, rewards2)
    return loss[0, 0], actions[:, 0]


# speedup vs baseline: 2.0887x; 1.0855x over previous
import jax
import jax.numpy as jnp
from jax.experimental import pallas as pl
from jax.experimental.pallas import tpu as pltpu

BATCH_ = 32
VOCAB_ = 1_000_000
VBLK = 16_384
GRID = -(-VOCAB_ // VBLK)

_NEG_INF = float("-inf")


def _body(logits_ref, gumbel_ref, rewards_ref,
          loss_ref, actions_ref,
          m_ref, acc_ref, gm_ref, gi_ref, gl_ref):
    j = pl.program_id(0)

    @pl.when(j == 0)
    def _init():
        m_ref[...] = jnp.full((BATCH_, 1), _NEG_INF, jnp.float32)
        acc_ref[...] = jnp.zeros((BATCH_, 1), jnp.float32)
        gm_ref[...] = jnp.full((BATCH_, 1), _NEG_INF, jnp.float32)
        gi_ref[...] = jnp.zeros((BATCH_, 1), jnp.int32)
        gl_ref[...] = jnp.zeros((BATCH_, 1), jnp.float32)

    def _update(x, u):
        iota = jax.lax.broadcasted_iota(jnp.int32, (BATCH_, VBLK), 1)
        bm = jnp.max(x, axis=1, keepdims=True)
        m_old = m_ref[...]
        m_new = jnp.maximum(m_old, bm)
        acc_ref[...] = (acc_ref[...] * jnp.exp(m_old - m_new)
                        + jnp.sum(jnp.exp(x - m_new), axis=1, keepdims=True))
        m_ref[...] = m_new
        s = x - jnp.log(-jnp.log(u))
        lm = jnp.max(s, axis=1, keepdims=True)
        big = jnp.int32(2**31 - 1)
        li = jnp.min(jnp.where(s == lm, iota, big), axis=1, keepdims=True)
        lx = jnp.sum(jnp.where(iota == li, x, 0.0), axis=1, keepdims=True)
        better = lm > gm_ref[...]
        gi_ref[...] = jnp.where(better, j * VBLK + li, gi_ref[...])
        gl_ref[...] = jnp.where(better, lx, gl_ref[...])
        gm_ref[...] = jnp.maximum(gm_ref[...], lm)

    @pl.when(j < GRID - 1)
    def _interior():
        _update(logits_ref[...], gumbel_ref[...])

    @pl.when(j == GRID - 1)
    def _tail():
        iota = jax.lax.broadcasted_iota(jnp.int32, (BATCH_, VBLK), 1)
        valid = (j * VBLK + iota) < VOCAB_
        _update(jnp.where(valid, logits_ref[...], _NEG_INF),
                jnp.where(valid, gumbel_ref[...], 0.5))
        lse = m_ref[...] + jnp.log(acc_ref[...])
        log_p = gl_ref[...] - lse
        r = rewards_ref[...]
        loss_ref[...] = jnp.sum(-log_p * r, keepdims=True).reshape(1, 1) / BATCH_
        actions_ref[...] = gi_ref[...]


@jax.jit
def kernel(logits, gumbel_noise, rewards):
    rewards2 = rewards.reshape(BATCH_, 1)
    loss, actions = pl.pallas_call(
        _body,
        grid=(GRID,),
        in_specs=[
            pl.BlockSpec((BATCH_, VBLK), lambda j: (0, j)),
            pl.BlockSpec((BATCH_, VBLK), lambda j: (0, j)),
            pl.BlockSpec((BATCH_, 1), lambda j: (0, 0)),
        ],
        out_specs=[
            pl.BlockSpec((1, 1), lambda j: (0, 0)),
            pl.BlockSpec((BATCH_, 1), lambda j: (0, 0)),
        ],
        out_shape=[
            jax.ShapeDtypeStruct((1, 1), jnp.float32),
            jax.ShapeDtypeStruct((BATCH_, 1), jnp.int32),
        ],
        scratch_shapes=[
            pltpu.VMEM((BATCH_, 1), jnp.float32),
            pltpu.VMEM((BATCH_, 1), jnp.float32),
            pltpu.VMEM((BATCH_, 1), jnp.float32),
            pltpu.VMEM((BATCH_, 1), jnp.int32),
            pltpu.VMEM((BATCH_, 1), jnp.float32),
        ],
    )(logits, gumbel_noise, rewards2)
    return loss[0, 0], actions[:, 0]
